# baseline (speedup 1.0000x reference)
import functools

import jax
import jax.numpy as jnp
import numpy as np
from jax import lax
from jax.experimental import pallas as pl
from jax.experimental.pallas import tpu as pltpu
from jax.experimental.pallas import tpu_sc as plsc

_VOCAB_SIZES = [100003, 100019, 100043, 100049, 100057, 100069, 100103, 100109]
_OFFSETS = np.cumsum([0] + _VOCAB_SIZES[:-1]).astype(np.int32)

_NUM_CORES = 2
_NUM_SUBCORES = 16
_NUM_WORKERS = _NUM_CORES * _NUM_SUBCORES
_LANES = 16
_GATHER_CHUNK = 128


@functools.partial(jax.jit, static_argnames=("n", "d"))
def _mhe_lookup(flat_ids, offsets16, table, *, n, d):
    n_per_w = n // _NUM_WORKERS
    n_chunks = n_per_w // _GATHER_CHUNK
    mesh = plsc.VectorSubcoreMesh(core_axis_name="c", subcore_axis_name="s")

    @functools.partial(
        pl.kernel,
        mesh=mesh,
        out_type=jax.ShapeDtypeStruct((n, d), jnp.float32),
        scratch_types=[
            pltpu.VMEM((n_per_w,), jnp.int32),
            pltpu.VMEM((_LANES,), jnp.int32),
            pltpu.VMEM((n_per_w, d), jnp.float32),
            pltpu.SemaphoreType.DMA,
        ],
        compiler_params=pltpu.CompilerParams(use_tc_tiling_on_sc=False),
    )
    def k(ids_hbm, off_hbm, table_hbm, out_hbm, idx_v, off_v, rows_v, sem):
        wid = lax.axis_index("s") * _NUM_CORES + lax.axis_index("c")
        base = wid * n_per_w
        pltpu.sync_copy(ids_hbm.at[pl.ds(base, n_per_w)], idx_v)
        pltpu.sync_copy(off_hbm, off_v)
        off = off_v[...]

        def shift_body(j, carry):
            sl = pl.ds(j * _LANES, _LANES)
            idx_v[sl] = idx_v[sl] + off
            return carry

        lax.fori_loop(0, n_per_w // _LANES, shift_body, 0)

        copies = []
        for c in range(n_chunks):
            sl = pl.ds(c * _GATHER_CHUNK, _GATHER_CHUNK)
            copies.append(
                pltpu.async_copy(table_hbm.at[idx_v.at[sl]], rows_v.at[sl], sem)
            )
        for cp in copies:
            cp.wait()
        pltpu.sync_copy(rows_v, out_hbm.at[pl.ds(base, n_per_w)])

    return k(flat_ids, offsets16, table)


def kernel(input_ids, table):
    b, s, h = input_ids.shape
    d = table.shape[1]
    n = b * s * h
    flat_ids = input_ids.reshape(n)
    offsets16 = jnp.asarray(np.tile(_OFFSETS, _LANES // len(_OFFSETS)))
    out = _mhe_lookup(flat_ids, offsets16, table, n=n, d=d)
    return out.reshape(b, s, h, d)
